# Initial kernel scaffold; baseline (speedup 1.0000x reference)
import jax
import jax.numpy as jnp
from jax.experimental import pallas as pl


def kernel(inputs, gate_kernel, w0_kernel, w1_kernel, wo_kernel):
    raise NotImplementedError("write your pallas kernel here")


# speedup vs baseline: 1.5210x; 1.5210x over previous
import functools

import jax
import jax.numpy as jnp
from jax.experimental import pallas as pl
from jax.experimental.pallas import tpu as pltpu

B, L, E = 1, 2048, 768
N_EXPERTS = 8
TOP_K = 2
MLP_DIM = 2048
H_BLK = 512


def _moe_kernel(x_ref, gate_ref, w0_ref, w1_ref, wo_ref, out_ref, wts_ref):
    n = pl.program_id(0)
    h = pl.program_id(1)

    @pl.when((n == 0) & (h == 0))
    def _prologue():
        logits = jnp.dot(x_ref[...], gate_ref[...],
                         preferred_element_type=jnp.float32)
        lane = jax.lax.broadcasted_iota(jnp.int32, logits.shape, 1)
        a1 = jnp.argmax(logits, axis=-1)[:, None]
        m1 = jnp.max(logits, axis=-1, keepdims=True)
        masked = jnp.where(lane == a1, -jnp.inf, logits)
        a2 = jnp.argmax(masked, axis=-1)[:, None]
        m2 = jnp.max(masked, axis=-1, keepdims=True)
        e2 = jnp.exp(m2 - m1)
        denom = 1.0 + e2
        p1 = 1.0 / denom
        p2 = e2 / denom
        wts = (jnp.where(lane == a1, p1, 0.0)
               + jnp.where(lane == a2, p2, 0.0))
        wts_ref[...] = wts
        out_ref[...] = jnp.zeros_like(out_ref)

    x = x_ref[...]
    h0 = jnp.dot(x, w0_ref[0], preferred_element_type=jnp.float32)
    h1 = jnp.dot(x, w1_ref[0], preferred_element_type=jnp.float32)
    m = (h0 * jax.nn.sigmoid(h0)) * h1
    y = jnp.dot(m, wo_ref[0], preferred_element_type=jnp.float32)
    lane = jax.lax.broadcasted_iota(jnp.int32, (L, N_EXPERTS), 1)
    wcol = jnp.sum(jnp.where(lane == n, wts_ref[...], 0.0),
                   axis=-1, keepdims=True)
    out_ref[...] += wcol * y


@functools.partial(jax.jit, static_argnames=())
def _moe(inputs, gate_kernel, w0_kernel, w1_kernel, wo_kernel):
    x = inputs.reshape(L, E).astype(jnp.float32)
    n_h = MLP_DIM // H_BLK
    out = pl.pallas_call(
        _moe_kernel,
        grid=(N_EXPERTS, n_h),
        in_specs=[
            pl.BlockSpec((L, E), lambda n, h: (0, 0)),
            pl.BlockSpec((E, N_EXPERTS), lambda n, h: (0, 0)),
            pl.BlockSpec((1, E, H_BLK), lambda n, h: (n, 0, h)),
            pl.BlockSpec((1, E, H_BLK), lambda n, h: (n, 0, h)),
            pl.BlockSpec((1, H_BLK, E), lambda n, h: (n, h, 0)),
        ],
        out_specs=pl.BlockSpec((L, E), lambda n, h: (0, 0)),
        out_shape=jax.ShapeDtypeStruct((L, E), jnp.float32),
        scratch_shapes=[pltpu.VMEM((L, N_EXPERTS), jnp.float32)],
        compiler_params=pltpu.CompilerParams(
            dimension_semantics=("arbitrary", "arbitrary"),
        ),
    )(x, gate_kernel, w0_kernel, w1_kernel, wo_kernel)
    return out.reshape(B, L, E)


def kernel(inputs, gate_kernel, w0_kernel, w1_kernel, wo_kernel):
    return _moe(inputs, gate_kernel, w0_kernel, w1_kernel, wo_kernel)
